# Initial kernel scaffold; baseline (speedup 1.0000x reference)
import functools

import jax
import jax.numpy as jnp
from jax import lax
from jax.experimental import pallas as pl
from jax.experimental.pallas import tpu as pltpu
from jax.experimental.pallas import tpu_sc as plsc

N_NODES = 10000
D_FEAT = 128
D_HID = 16
N_EDGES = 320000

NC = 2
NS = 16
NW = NC * NS
CHUNK = 128
NCH = 79
E_PAD = NW * NCH * CHUNK
NP = 10016
ROWS_PER_SUB = NP // NS


def _vector_mesh():
    return plsc.VectorSubcoreMesh(
        core_axis_name="c", subcore_axis_name="s", num_cores=NC, num_subcores=NS
    )


def _sc_degree(dstb, ones_hbm, zeros_hbm):
    @functools.partial(
        pl.kernel,
        out_type=jax.ShapeDtypeStruct((NC, NP, D_HID), jnp.float32),
        mesh=_vector_mesh(),
        scratch_types=[
            pltpu.VMEM((NCH, CHUNK), jnp.int32),
            pltpu.VMEM((CHUNK, D_HID), jnp.float32),
            pltpu.VMEM_SHARED((NP, D_HID), jnp.float32),
        ],
    )
    def deg_kernel(dstb_hbm, ones_h, zeros_h, out_hbm, didx_v, ones_v, acc_s):
        c = lax.axis_index("c")
        s = lax.axis_index("s")
        wid = s * NC + c
        r0 = s * ROWS_PER_SUB
        pltpu.sync_copy(zeros_h.at[pl.ds(r0, ROWS_PER_SUB)],
                        acc_s.at[pl.ds(r0, ROWS_PER_SUB)])
        pltpu.sync_copy(ones_h, ones_v)
        pltpu.sync_copy(dstb_hbm.at[wid], didx_v)
        plsc.subcore_barrier()

        @pl.loop(0, NCH)
        def _(j):
            pltpu.sync_copy(ones_v, acc_s.at[didx_v.at[j]], add=True)

        plsc.subcore_barrier()
        pltpu.sync_copy(acc_s.at[pl.ds(r0, ROWS_PER_SUB)],
                        out_hbm.at[c].at[pl.ds(r0, ROWS_PER_SUB)])

    return deg_kernel(dstb, ones_hbm, zeros_hbm)


def _sc_aggregate(hs, srcb, dstb, zeros_hbm):
    @functools.partial(
        pl.kernel,
        out_type=jax.ShapeDtypeStruct((NC, NP, D_HID), jnp.float32),
        mesh=_vector_mesh(),
        scratch_types=[
            pltpu.VMEM((NCH, CHUNK), jnp.int32),
            pltpu.VMEM((NCH, CHUNK), jnp.int32),
            pltpu.VMEM((CHUNK, D_HID), jnp.float32),
            pltpu.VMEM_SHARED((NP, D_HID), jnp.float32),
            pltpu.VMEM_SHARED((NP, D_HID), jnp.float32),
            pltpu.SemaphoreType.DMA,
        ],
    )
    def agg_kernel(hs_hbm, srcb_hbm, dstb_hbm, zeros_h, out_hbm,
                   sidx_v, didx_v, rows_v, hs_s, acc_s, sem):
        c = lax.axis_index("c")
        s = lax.axis_index("s")
        wid = s * NC + c
        r0 = s * ROWS_PER_SUB
        pltpu.sync_copy(hs_hbm.at[pl.ds(r0, ROWS_PER_SUB)],
                        hs_s.at[pl.ds(r0, ROWS_PER_SUB)])
        pltpu.sync_copy(zeros_h.at[pl.ds(r0, ROWS_PER_SUB)],
                        acc_s.at[pl.ds(r0, ROWS_PER_SUB)])
        pltpu.sync_copy(srcb_hbm.at[wid], sidx_v)
        pltpu.sync_copy(dstb_hbm.at[wid], didx_v)
        plsc.subcore_barrier()

        @pl.loop(0, NCH)
        def _(j):
            pltpu.async_copy(hs_s.at[sidx_v.at[j]], rows_v, sem).wait()
            pltpu.sync_copy(rows_v, acc_s.at[didx_v.at[j]], add=True)

        plsc.subcore_barrier()
        pltpu.sync_copy(acc_s.at[pl.ds(r0, ROWS_PER_SUB)],
                        out_hbm.at[c].at[pl.ds(r0, ROWS_PER_SUB)])

    return agg_kernel(hs, srcb, dstb, zeros_hbm)


def _tc_call(body, out_shape, *args):
    return pl.pallas_call(
        body,
        out_shape=jax.ShapeDtypeStruct(out_shape, jnp.float32),
    )(*args)


def _mm1_body(x_ref, g_ref, w_ref, o_ref):
    xf = x_ref[...] * g_ref[...]
    o_ref[...] = jnp.dot(xf, w_ref[...], preferred_element_type=jnp.float32)


def _scale_body(degp_ref, h1_ref, dinv_ref, hs_ref):
    deg = degp_ref[0] + degp_ref[1] + 1.0
    dinv = lax.rsqrt(deg)
    dinv_ref[...] = dinv
    hs_ref[...] = h1_ref[...] * dinv


def _relu_body(aggp_ref, hs1_ref, dinv_ref, b1_ref, hs2_ref):
    dinv = dinv_ref[...]
    conv1 = dinv * (aggp_ref[0] + aggp_ref[1] + hs1_ref[...]) + b1_ref[...]
    hs2_ref[...] = jnp.maximum(conv1, 0.0) * dinv


def _mm2_body(aggp_ref, hs2_ref, dinv_ref, w_ref, b_ref, o_ref):
    agg2 = dinv_ref[...] * (aggp_ref[0] + aggp_ref[1] + hs2_ref[...])
    o_ref[...] = (
        jnp.dot(agg2, w_ref[...], preferred_element_type=jnp.float32)
        + b_ref[...]
    )


def kernel(x, graph_seq, edge_index, W1, b1, W2, b2):
    nb, nc_, nd = x.shape
    n = nb * nc_

    xf = x.reshape(n, nd)
    gs = graph_seq.reshape(n, 1)
    pad_n = NP - n
    xf = jnp.concatenate([xf, jnp.zeros((pad_n, nd), jnp.float32)], axis=0)
    gs = jnp.concatenate([gs, jnp.zeros((pad_n, 1), jnp.float32)], axis=0)

    src = edge_index[0].astype(jnp.int32)
    dst = edge_index[1].astype(jnp.int32)
    pad_e = E_PAD - N_EDGES
    src_p = jnp.concatenate([src, jnp.zeros((pad_e,), jnp.int32)])
    dst_p = jnp.concatenate([dst, jnp.full((pad_e,), N_NODES, jnp.int32)])
    srcb = src_p.reshape(NW, NCH, CHUNK)
    dstb = dst_p.reshape(NW, NCH, CHUNK)

    zeros = jnp.zeros((NP, D_HID), jnp.float32)
    ones = jnp.ones((CHUNK, D_HID), jnp.float32)
    b1r = b1.reshape(1, D_HID)
    b2r = b2.reshape(1, D_FEAT)

    degp = _sc_degree(dstb, ones, zeros)

    h1 = _tc_call(_mm1_body, (NP, D_HID), xf, gs, W1)

    dinv, hs1 = pl.pallas_call(
        _scale_body,
        out_shape=(
            jax.ShapeDtypeStruct((NP, D_HID), jnp.float32),
            jax.ShapeDtypeStruct((NP, D_HID), jnp.float32),
        ),
    )(degp, h1)

    agg1 = _sc_aggregate(hs1, srcb, dstb, zeros)

    hs2 = _tc_call(_relu_body, (NP, D_HID), agg1, hs1, dinv, b1r)

    agg2 = _sc_aggregate(hs2, srcb, dstb, zeros)

    out = _tc_call(_mm2_body, (NP, D_FEAT), agg2, hs2, dinv, W2, b2r)

    return out[:n].reshape(nb, nc_, nd)


# speedup vs baseline: 43.2489x; 43.2489x over previous
import functools

import jax
import jax.numpy as jnp
from jax import lax
from jax.experimental import pallas as pl
from jax.experimental.pallas import tpu as pltpu
from jax.experimental.pallas import tpu_sc as plsc

N_NODES = 10000
D_FEAT = 128
D_HID = 16
N_EDGES = 320000

NC = 2
NS = 16
NW = NC * NS
CHUNK = 128
NCH = 79
E_PAD = NW * NCH * CHUNK
NP = 10112
ROWS_PER_SUB = NP // NS


def _vector_mesh():
    return plsc.VectorSubcoreMesh(
        core_axis_name="c", subcore_axis_name="s", num_cores=NC, num_subcores=NS
    )


def _sc_degree(dstb, ones_hbm, zeros_hbm):
    @functools.partial(
        pl.kernel,
        out_type=jax.ShapeDtypeStruct((NC, NP, D_HID), jnp.float32),
        mesh=_vector_mesh(),
        scratch_types=[
            pltpu.VMEM((NCH, CHUNK), jnp.int32),
            pltpu.VMEM((CHUNK, D_HID), jnp.float32),
            pltpu.VMEM_SHARED((NP, D_HID), jnp.float32),
        ],
    )
    def deg_kernel(dstb_hbm, ones_h, zeros_h, out_hbm, didx_v, ones_v, acc_s):
        c = lax.axis_index("c")
        s = lax.axis_index("s")
        wid = s * NC + c
        r0 = s * ROWS_PER_SUB
        pltpu.sync_copy(zeros_h.at[pl.ds(r0, ROWS_PER_SUB)],
                        acc_s.at[pl.ds(r0, ROWS_PER_SUB)])
        pltpu.sync_copy(ones_h, ones_v)
        pltpu.sync_copy(dstb_hbm.at[wid], didx_v)
        plsc.subcore_barrier()

        @pl.loop(0, NCH)
        def _(j):
            pltpu.sync_copy(ones_v, acc_s.at[didx_v.at[j]], add=True)

        plsc.subcore_barrier()
        pltpu.sync_copy(acc_s.at[pl.ds(r0, ROWS_PER_SUB)],
                        out_hbm.at[c].at[pl.ds(r0, ROWS_PER_SUB)])

    return deg_kernel(dstb, ones_hbm, zeros_hbm)


def _sc_aggregate(hs, srcb, dstb, zeros_hbm):
    @functools.partial(
        pl.kernel,
        out_type=jax.ShapeDtypeStruct((NC, NP, D_HID), jnp.float32),
        mesh=_vector_mesh(),
        scratch_types=[
            pltpu.VMEM((NCH, CHUNK), jnp.int32),
            pltpu.VMEM((NCH, CHUNK), jnp.int32),
            pltpu.VMEM((CHUNK, D_HID), jnp.float32),
            pltpu.VMEM_SHARED((NP, D_HID), jnp.float32),
            pltpu.VMEM_SHARED((NP, D_HID), jnp.float32),
            pltpu.SemaphoreType.DMA,
        ],
    )
    def agg_kernel(hs_hbm, srcb_hbm, dstb_hbm, zeros_h, out_hbm,
                   sidx_v, didx_v, rows_v, hs_s, acc_s, sem):
        c = lax.axis_index("c")
        s = lax.axis_index("s")
        wid = s * NC + c
        r0 = s * ROWS_PER_SUB
        pltpu.sync_copy(hs_hbm.at[pl.ds(r0, ROWS_PER_SUB)],
                        hs_s.at[pl.ds(r0, ROWS_PER_SUB)])
        pltpu.sync_copy(zeros_h.at[pl.ds(r0, ROWS_PER_SUB)],
                        acc_s.at[pl.ds(r0, ROWS_PER_SUB)])
        pltpu.sync_copy(srcb_hbm.at[wid], sidx_v)
        pltpu.sync_copy(dstb_hbm.at[wid], didx_v)
        plsc.subcore_barrier()

        @pl.loop(0, NCH)
        def _(j):
            pltpu.async_copy(hs_s.at[sidx_v.at[j]], rows_v, sem).wait()
            pltpu.sync_copy(rows_v, acc_s.at[didx_v.at[j]], add=True)

        plsc.subcore_barrier()
        pltpu.sync_copy(acc_s.at[pl.ds(r0, ROWS_PER_SUB)],
                        out_hbm.at[c].at[pl.ds(r0, ROWS_PER_SUB)])

    return agg_kernel(hs, srcb, dstb, zeros_hbm)


def _tc_call(body, out_shape, *args):
    return pl.pallas_call(
        body,
        out_shape=jax.ShapeDtypeStruct(out_shape, jnp.float32),
    )(*args)


def _mm1_body(x_ref, g_ref, w_ref, o_ref):
    xf = x_ref[...] * g_ref[...]
    o_ref[...] = jnp.dot(xf, w_ref[...], preferred_element_type=jnp.float32)


def _scale_body(degp_ref, h1_ref, dinv_ref, hs_ref):
    deg = degp_ref[0] + degp_ref[1] + 1.0
    dinv = lax.rsqrt(deg)
    dinv_ref[...] = dinv
    hs_ref[...] = h1_ref[...] * dinv


def _relu_body(aggp_ref, hs1_ref, dinv_ref, b1_ref, hs2_ref):
    dinv = dinv_ref[...]
    conv1 = dinv * (aggp_ref[0] + aggp_ref[1] + hs1_ref[...]) + b1_ref[...]
    hs2_ref[...] = jnp.maximum(conv1, 0.0) * dinv


def _mm2_body(aggp_ref, hs2_ref, dinv_ref, w_ref, b_ref, o_ref):
    agg2 = dinv_ref[...] * (aggp_ref[0] + aggp_ref[1] + hs2_ref[...])
    o_ref[...] = (
        jnp.dot(agg2, w_ref[...], preferred_element_type=jnp.float32)
        + b_ref[...]
    )


def kernel(x, graph_seq, edge_index, W1, b1, W2, b2):
    nb, nc_, nd = x.shape
    n = nb * nc_

    xf = x.reshape(n, nd)
    gs = graph_seq.reshape(n, 1)
    pad_n = NP - n
    xf = jnp.concatenate([xf, jnp.zeros((pad_n, nd), jnp.float32)], axis=0)
    gs = jnp.concatenate([gs, jnp.zeros((pad_n, 1), jnp.float32)], axis=0)

    src = edge_index[0].astype(jnp.int32)
    dst = edge_index[1].astype(jnp.int32)
    pad_e = E_PAD - N_EDGES
    src_p = jnp.concatenate([src, jnp.zeros((pad_e,), jnp.int32)])
    dst_p = jnp.concatenate([dst, jnp.full((pad_e,), N_NODES, jnp.int32)])
    srcb = src_p.reshape(NW, NCH, CHUNK)
    dstb = dst_p.reshape(NW, NCH, CHUNK)

    zeros = jnp.zeros((NP, D_HID), jnp.float32)
    ones = jnp.ones((CHUNK, D_HID), jnp.float32)
    b1r = b1.reshape(1, D_HID)
    b2r = b2.reshape(1, D_FEAT)

    degp = _sc_degree(dstb, ones, zeros)

    h1 = _tc_call(_mm1_body, (NP, D_HID), xf, gs, W1)

    dinv, hs1 = pl.pallas_call(
        _scale_body,
        out_shape=(
            jax.ShapeDtypeStruct((NP, D_HID), jnp.float32),
            jax.ShapeDtypeStruct((NP, D_HID), jnp.float32),
        ),
    )(degp, h1)

    agg1 = _sc_aggregate(hs1, srcb, dstb, zeros)

    hs2 = _tc_call(_relu_body, (NP, D_HID), agg1, hs1, dinv, b1r)

    agg2 = _sc_aggregate(hs2, srcb, dstb, zeros)

    out = _tc_call(_mm2_body, (NP, D_FEAT), agg2, hs2, dinv, W2, b2r)

    return out[:n].reshape(nb, nc_, nd)


# speedup vs baseline: 48.6826x; 1.1256x over previous
import functools

import jax
import jax.numpy as jnp
from jax import lax
from jax.experimental import pallas as pl
from jax.experimental.pallas import tpu as pltpu
from jax.experimental.pallas import tpu_sc as plsc

N_NODES = 10000
D_FEAT = 128
D_HID = 16
N_EDGES = 320000

NC = 2
NS = 16
NW = NC * NS
CHUNK = 128
NCH = 80
E_PAD = NW * NCH * CHUNK
NP = 10112
ROWS_PER_SUB = NP // NS
G = 4
NGRP = NCH // G


def _vector_mesh():
    return plsc.VectorSubcoreMesh(
        core_axis_name="c", subcore_axis_name="s", num_cores=NC, num_subcores=NS
    )


def _sc_degree(dstb, ones_hbm, zeros_hbm):
    @functools.partial(
        pl.kernel,
        out_type=jax.ShapeDtypeStruct((NC, NP, D_HID), jnp.float32),
        mesh=_vector_mesh(),
        compiler_params=pltpu.CompilerParams(use_tc_tiling_on_sc=False),
        scratch_types=[
            pltpu.VMEM((NCH, CHUNK), jnp.int32),
            pltpu.VMEM((CHUNK, D_HID), jnp.float32),
            pltpu.VMEM_SHARED((NP, D_HID), jnp.float32),
            pltpu.SemaphoreType.DMA,
        ],
    )
    def deg_kernel(dstb_hbm, ones_h, zeros_h, out_hbm, didx_v, ones_v, acc_s,
                   sem):
        c = lax.axis_index("c")
        s = lax.axis_index("s")
        wid = s * NC + c
        r0 = s * ROWS_PER_SUB
        pltpu.sync_copy(zeros_h.at[pl.ds(r0, ROWS_PER_SUB)],
                        acc_s.at[pl.ds(r0, ROWS_PER_SUB)])
        pltpu.sync_copy(ones_h, ones_v)
        pltpu.sync_copy(dstb_hbm.at[wid], didx_v)
        plsc.subcore_barrier()

        def fire8(g):
            for k in range(8):
                pltpu.async_copy(ones_v, acc_s.at[didx_v.at[g * 8 + k]], sem,
                                 add=True)

        def drain(n):
            for _ in range(n):
                pltpu.make_async_copy(zeros_h.at[pl.ds(0, CHUNK)], ones_v,
                                      sem).wait()

        fire8(0)

        @pl.loop(0, NCH // 8 - 1)
        def _(g):
            fire8(g + 1)
            drain(8)

        drain(8)
        plsc.subcore_barrier()
        pltpu.sync_copy(acc_s.at[pl.ds(r0, ROWS_PER_SUB)],
                        out_hbm.at[c].at[pl.ds(r0, ROWS_PER_SUB)])

    return deg_kernel(dstb, ones_hbm, zeros_hbm)


def _sc_aggregate(hs, srcb, dstb, zeros_hbm):
    @functools.partial(
        pl.kernel,
        out_type=jax.ShapeDtypeStruct((NC, NP, D_HID), jnp.float32),
        mesh=_vector_mesh(),
        compiler_params=pltpu.CompilerParams(use_tc_tiling_on_sc=False),
        scratch_types=[
            pltpu.VMEM((NCH, CHUNK), jnp.int32),
            pltpu.VMEM((NCH, CHUNK), jnp.int32),
            pltpu.VMEM((G, CHUNK, D_HID), jnp.float32),
            pltpu.VMEM((G, CHUNK, D_HID), jnp.float32),
            pltpu.VMEM_SHARED((NP, D_HID), jnp.float32),
            pltpu.VMEM_SHARED((NP, D_HID), jnp.float32),
            pltpu.SemaphoreType.DMA,
            pltpu.SemaphoreType.DMA,
            pltpu.SemaphoreType.DMA,
            pltpu.SemaphoreType.DMA,
        ],
    )
    def agg_kernel(hs_hbm, srcb_hbm, dstb_hbm, zeros_h, out_hbm,
                   sidx_v, didx_v, rows_a, rows_b, hs_s, acc_s,
                   semga, semgb, semsa, semsb):
        c = lax.axis_index("c")
        s = lax.axis_index("s")
        wid = s * NC + c
        r0 = s * ROWS_PER_SUB
        pltpu.sync_copy(hs_hbm.at[pl.ds(r0, ROWS_PER_SUB)],
                        hs_s.at[pl.ds(r0, ROWS_PER_SUB)])
        pltpu.sync_copy(zeros_h.at[pl.ds(r0, ROWS_PER_SUB)],
                        acc_s.at[pl.ds(r0, ROWS_PER_SUB)])
        pltpu.sync_copy(srcb_hbm.at[wid], sidx_v)
        pltpu.sync_copy(dstb_hbm.at[wid], didx_v)
        plsc.subcore_barrier()

        def fire_g(g, rows, sem):
            for k in range(G):
                pltpu.async_copy(hs_s.at[sidx_v.at[g * G + k]], rows.at[k],
                                 sem)

        def fire_s(g, rows, sem):
            for k in range(G):
                pltpu.async_copy(rows.at[k], acc_s.at[didx_v.at[g * G + k]],
                                 sem, add=True)

        def drain(sem, n):
            for _ in range(n):
                pltpu.make_async_copy(zeros_h.at[pl.ds(0, CHUNK)],
                                      rows_a.at[0], sem).wait()

        fire_g(0, rows_a, semga)
        fire_g(1, rows_b, semgb)
        drain(semga, G)
        fire_s(0, rows_a, semsa)

        @pl.loop(1, NGRP - 1, step=2)
        def _(g):
            drain(semsa, G)
            fire_g(g + 1, rows_a, semga)
            drain(semgb, G)
            fire_s(g, rows_b, semsb)
            drain(semsb, G)
            fire_g(g + 2, rows_b, semgb)
            drain(semga, G)
            fire_s(g + 1, rows_a, semsa)

        drain(semsa, G)
        drain(semgb, G)
        fire_s(NGRP - 1, rows_b, semsb)
        drain(semsb, G)

        plsc.subcore_barrier()
        pltpu.sync_copy(acc_s.at[pl.ds(r0, ROWS_PER_SUB)],
                        out_hbm.at[c].at[pl.ds(r0, ROWS_PER_SUB)])

    return agg_kernel(hs, srcb, dstb, zeros_hbm)


def _tc_call(body, out_shape, *args):
    return pl.pallas_call(
        body,
        out_shape=jax.ShapeDtypeStruct(out_shape, jnp.float32),
    )(*args)


def _mm1_scale_body(x_ref, g_ref, w_ref, degp_ref, dinv_ref, hs_ref):
    xf = x_ref[...] * g_ref[...]
    h1 = jnp.dot(xf, w_ref[...], preferred_element_type=jnp.float32)
    deg = degp_ref[0] + degp_ref[1] + 1.0
    dinv = lax.rsqrt(deg)
    dinv_ref[...] = dinv
    hs_ref[...] = h1 * dinv


def _relu_body(aggp_ref, hs1_ref, dinv_ref, b1_ref, hs2_ref):
    dinv = dinv_ref[...]
    conv1 = dinv * (aggp_ref[0] + aggp_ref[1] + hs1_ref[...]) + b1_ref[...]
    hs2_ref[...] = jnp.maximum(conv1, 0.0) * dinv


def _mm2_body(aggp_ref, hs2_ref, dinv_ref, w_ref, b_ref, o_ref):
    agg2 = dinv_ref[...] * (aggp_ref[0] + aggp_ref[1] + hs2_ref[...])
    o_ref[...] = (
        jnp.dot(agg2, w_ref[...], preferred_element_type=jnp.float32)
        + b_ref[...]
    )


def kernel(x, graph_seq, edge_index, W1, b1, W2, b2):
    nb, nc_, nd = x.shape
    n = nb * nc_

    xf = x.reshape(n, nd)
    gs = graph_seq.reshape(n, 1)
    pad_n = NP - n
    xf = jnp.concatenate([xf, jnp.zeros((pad_n, nd), jnp.float32)], axis=0)
    gs = jnp.concatenate([gs, jnp.zeros((pad_n, 1), jnp.float32)], axis=0)

    src = edge_index[0].astype(jnp.int32)
    dst = edge_index[1].astype(jnp.int32)
    pad_e = E_PAD - N_EDGES
    src_p = jnp.concatenate([src, jnp.zeros((pad_e,), jnp.int32)])
    dst_p = jnp.concatenate([dst, jnp.full((pad_e,), N_NODES, jnp.int32)])
    srcb = src_p.reshape(NW, NCH, CHUNK)
    dstb = dst_p.reshape(NW, NCH, CHUNK)

    zeros = jnp.zeros((NP, D_HID), jnp.float32)
    ones = jnp.ones((CHUNK, D_HID), jnp.float32)
    b1r = b1.reshape(1, D_HID)
    b2r = b2.reshape(1, D_FEAT)

    degp = _sc_degree(dstb, ones, zeros)

    dinv, hs1 = pl.pallas_call(
        _mm1_scale_body,
        out_shape=(
            jax.ShapeDtypeStruct((NP, D_HID), jnp.float32),
            jax.ShapeDtypeStruct((NP, D_HID), jnp.float32),
        ),
    )(xf, gs, W1, degp)

    agg1 = _sc_aggregate(hs1, srcb, dstb, zeros)

    hs2 = _tc_call(_relu_body, (NP, D_HID), agg1, hs1, dinv, b1r)

    agg2 = _sc_aggregate(hs2, srcb, dstb, zeros)

    out = _tc_call(_mm2_body, (NP, D_FEAT), agg2, hs2, dinv, W2, b2r)

    return out[:n].reshape(nb, nc_, nd)
